# Initial kernel scaffold; baseline (speedup 1.0000x reference)
import jax
import jax.numpy as jnp
from jax.experimental import pallas as pl


def kernel(output):
    raise NotImplementedError("write your pallas kernel here")


# speedup vs baseline: 1.5174x; 1.5174x over previous
import functools

import jax
import jax.numpy as jnp
import numpy as np
from jax import lax
from jax.experimental import pallas as pl

_NUM_CLASSES = 80
_C = _NUM_CLASSES + 6
_G = 64
_GG = _G * _G
_NA = 18
_SCALE_XY = 1.05
_STRIDE = 8.0
_ANCHORS_W = np.array([12.0, 19.0, 40.0], dtype=np.float32)
_ANCHORS_H = np.array([16.0, 36.0, 28.0], dtype=np.float32)
_PI6 = 0.5235987755982988


def _body(x_ref, o_ref):
    ba = pl.program_id(0)
    a = ba % _NA
    ai = a // 6
    aj = a % 6
    aw8 = jnp.where(ai == 0, 12.0, jnp.where(ai == 1, 19.0, 40.0))
    ah8 = jnp.where(ai == 0, 16.0, jnp.where(ai == 1, 36.0, 28.0))
    aa = (aj.astype(jnp.float32) - 2.0) * np.float32(_PI6)

    v = x_ref[0]
    ci = lax.broadcasted_iota(jnp.int32, (_C, _GG), 0)
    gi = lax.broadcasted_iota(jnp.int32, (_C, _GG), 1)
    gx = (gi % _G).astype(jnp.float32)
    gy = (gi // _G).astype(jnp.float32)
    s = 1.0 / (1.0 + jnp.exp(-v))
    ex = jnp.exp(v)
    sxy = s * np.float32(_SCALE_XY * _STRIDE)
    off = np.float32((_SCALE_XY - 1.0) / 2.0 * _STRIDE)
    r = jnp.where(
        ci == 0, sxy + gx * np.float32(_STRIDE) - off,
        jnp.where(
            ci == 1, sxy + gy * np.float32(_STRIDE) - off,
            jnp.where(
                ci == 2, ex * aw8,
                jnp.where(ci == 3, ex * ah8, jnp.where(ci == 4, v + aa, s)))))
    o_ref[0] = r.T


def kernel(output):
    B = output.shape[0]
    ba_total = B * _NA
    x = output.reshape(ba_total, _C, _GG)
    out = pl.pallas_call(
        _body,
        grid=(ba_total,),
        in_specs=[pl.BlockSpec((1, _C, _GG), lambda i: (i, 0, 0))],
        out_specs=pl.BlockSpec((1, _GG, _C), lambda i: (i, 0, 0)),
        out_shape=jax.ShapeDtypeStruct((ba_total, _GG, _C), jnp.float32),
    )(x)
    return out.reshape(B, _NA * _GG, _C)
